# baseline (speedup 1.0000x reference)
import functools

import jax
import jax.numpy as jnp
from jax import lax
from jax.experimental import pallas as pl
from jax.experimental.pallas import tpu as pltpu
from jax.experimental.pallas import tpu_sc as plsc

_BITS = 48
_C = 100
_N = 50000
_BATCH = 128
_LAM = 1.0
_ETA_MU = 0.1
_T = 6400
_NT = -(-_N // _T)
_HI = lax.Precision.HIGHEST
_F32 = jnp.float32
_BF16 = jnp.bfloat16

_INTERPRET = False


def _eye(n):
    ii = lax.broadcasted_iota(jnp.int32, (n, n), 0)
    jj = lax.broadcasted_iota(jnp.int32, (n, n), 1)
    return (ii == jj).astype(_F32)


def _row_set(M, i, row):
    ii = lax.broadcasted_iota(jnp.int32, (M.shape[0], 1), 0)
    return jnp.where(ii == i, row, M)


def _gj_solve(A, R):
    M = jnp.concatenate([A, R], axis=1)
    for i in range(_BITS):
        piv = M[i:i + 1, i:i + 1]
        row = M[i:i + 1, :] / piv
        col = M[:, i:i + 1]
        M = M - col * row
        M = _row_set(M, i, row)
    return M[:, _BITS:]


def _zero_diag(G):
    return G * (1.0 - _eye(_BITS))


_BS = 8


def _bit_loop(B_scr, P, G0):
    G16 = G0.astype(_BF16)
    ii = lax.broadcasted_iota(jnp.int32, (_BS, _BS), 0)
    jj = lax.broadcasted_iota(jnp.int32, (_BS, _BS), 1)
    ltri = (ii > jj).astype(_F32)
    for b0 in range(0, _BITS, _BS):
        Qb = lax.dot_general(G16[b0:b0 + _BS, :], B_scr[...],
                             (((1,), (0,)), ((), ())),
                             preferred_element_type=_F32)
        X = P[b0:b0 + _BS, :] - Qb
        DB = G0[b0:b0 + _BS, b0:b0 + _BS] * ltri
        for li in range(_BS):
            i = b0 + li
            x = X[li:li + 1, :]
            old = B_scr[i:i + 1, :].astype(_F32)
            newb = jnp.where(x > 0.0, 1.0, -1.0)
            delta = newb - old
            B_scr[i:i + 1, :] = newb.astype(_BF16)
            if li < _BS - 1:
                X = X - DB[:, li:li + 1] * delta
    return B_scr[...]


def _onehot_from_labels(lab):
    cc = lax.broadcasted_iota(jnp.int32, (_C, 1), 0).astype(_F32)
    return (cc == lab).astype(_F32)


def _colmask(step, t):
    col = lax.broadcasted_iota(jnp.int32, (1, t), 1) + step * t
    return col < _N


def _k12_body(B_ref, Y_ref, U_ref, S1_ref, R1_ref, W1_ref,
              S0_scr, R0_scr, lab_scr, W_scr, G0_scr, B_scr):
    s = pl.program_id(0)

    @pl.when(s == 0)
    def _():
        S0_scr[...] = jnp.zeros_like(S0_scr)
        R0_scr[...] = jnp.zeros_like(R0_scr)

    @pl.when(s < _NT)
    def _():
        mask = _colmask(s, _T)
        bb = jnp.where(mask, B_ref[...], 0.0).astype(_BF16)
        yy = jnp.where(mask, Y_ref[...], 0.0).astype(_BF16)
        S0_scr[...] += lax.dot_general(bb, bb, (((1,), (1,)), ((), ())),
                                       preferred_element_type=_F32)
        R0_scr[...] += lax.dot_general(bb, yy, (((1,), (1,)), ((), ())),
                                       preferred_element_type=_F32)
        cvec = lax.broadcasted_iota(jnp.int32, (1, _C), 1).astype(_BF16)
        lab_scr[pl.ds(s, 1), :] = lax.dot_general(
            cvec, yy, (((1,), (0,)), ((), ())), preferred_element_type=_F32)

    @pl.when(s == _NT)
    def _():
        A = S0_scr[...] + _LAM * _eye(_BITS)
        W = _gj_solve(A, R0_scr[...])
        G = lax.dot_general(W, W, (((1,), (1,)), ((), ())), precision=_HI)
        G0_scr[...] = _zero_diag(G)
        W_scr[...] = W
        W1_ref[...] = W
        S1_ref[...] = jnp.zeros_like(S1_ref)
        R1_ref[...] = jnp.zeros_like(R1_ref)

    @pl.when(s >= _NT)
    def _():
        t = s - _NT
        W = W_scr[...]
        G0 = G0_scr[...]
        mask = _colmask(t, _T)
        lab = lab_scr[pl.ds(t, 1), :]
        oneh = jnp.where(mask, _onehot_from_labels(lab), 0.0)
        ob = oneh.astype(_BF16)
        P = lax.dot_general(W.astype(_BF16), ob, (((1,), (0,)), ((), ())),
                            preferred_element_type=_F32) + _ETA_MU * U_ref[...]
        B_scr[...] = B_ref[...].astype(_BF16)
        Bn = _bit_loop(B_scr, P, G0)
        bb = jnp.where(mask, Bn, jnp.bfloat16(0))
        S1_ref[...] += lax.dot_general(bb, bb, (((1,), (1,)), ((), ())),
                                       preferred_element_type=_F32)
        R1_ref[...] += lax.dot_general(bb, ob, (((1,), (1,)), ((), ())),
                                       preferred_element_type=_F32)


def _k12_call(B, Y, U):
    return pl.pallas_call(
        _k12_body,
        grid=(2 * _NT,),
        in_specs=[
            pl.BlockSpec((_BITS, _T), lambda s: (0, lax.rem(s, _NT))),
            pl.BlockSpec((_C, _T), lambda s: (0, jnp.minimum(s, _NT - 1))),
            pl.BlockSpec((_BITS, _T),
                         lambda s: (0, jnp.maximum(s - _NT, 0))),
        ],
        out_specs=[
            pl.BlockSpec((_BITS, _BITS), lambda s: (0, 0)),
            pl.BlockSpec((_BITS, _C), lambda s: (0, 0)),
            pl.BlockSpec((_BITS, _C), lambda s: (0, 0)),
        ],
        out_shape=[
            jax.ShapeDtypeStruct((_BITS, _BITS), _F32),
            jax.ShapeDtypeStruct((_BITS, _C), _F32),
            jax.ShapeDtypeStruct((_BITS, _C), _F32),
        ],
        scratch_shapes=[
            pltpu.VMEM((_BITS, _BITS), _F32),
            pltpu.VMEM((_BITS, _C), _F32),
            pltpu.VMEM((_NT, _T), _F32),
            pltpu.VMEM((_BITS, _C), _F32),
            pltpu.VMEM((_BITS, _BITS), _F32),
            pltpu.VMEM((_BITS, _T), _BF16),
        ],
        interpret=_INTERPRET,
    )(B, Y, U)


def _k3_body(Ub_ref, Yb_ref, S1_ref, R1_ref, W1_ref, Bi_ref, Ui_ref, Yi_ref,
             l_ref, sl_ref, cl_ref, ql_ref, B_scr):
    W1 = W1_ref[...]
    A = S1_ref[...] + _LAM * _eye(_BITS)
    W2 = _gj_solve(A, R1_ref[...])
    G10 = _zero_diag(lax.dot_general(W1, W1, (((1,), (1,)), ((), ())),
                                     precision=_HI))
    G20 = _zero_diag(lax.dot_general(W2, W2, (((1,), (1,)), ((), ())),
                                     precision=_HI))
    oneh = Yi_ref[...]
    Ui = Ui_ref[...]
    P1 = lax.dot_general(W1, oneh, (((1,), (0,)), ((), ())),
                         precision=_HI) + _ETA_MU * Ui
    B_scr[...] = Bi_ref[...].astype(_BF16)
    _bit_loop(B_scr, P1, G10)
    P2 = lax.dot_general(W2, oneh, (((1,), (0,)), ((), ())),
                         precision=_HI) + _ETA_MU * Ui
    B2 = _bit_loop(B_scr, P2, G20).astype(_F32)

    Ub = Ub_ref[...]
    Yb = Yb_ref[...]
    theta = 0.5 * lax.dot_general(Ub, Ub, (((1,), (1,)), ((), ())),
                                  precision=_HI)
    yb16 = Yb.astype(_BF16)
    Sm = (lax.dot_general(yb16, yb16, (((1,), (1,)), ((), ())),
                          preferred_element_type=_F32) > 0).astype(_F32)
    sp = jnp.maximum(theta, 0.0) + jnp.log(1.0 + jnp.exp(-jnp.abs(theta)))
    sim = jnp.mean(sp - Sm * theta)

    WB_T = lax.dot_general(B2, W2, (((0,), (0,)), ((), ())),
                           precision=_HI)
    cls = jnp.mean((Yb - WB_T) ** 2)
    qua = jnp.mean((Ub - jnp.transpose(B2)) ** 2)

    l_ref[...] = jnp.reshape(sim + 1.0 * cls + _ETA_MU * qua, (1, 1))
    sl_ref[...] = jnp.reshape(sim, (1, 1))
    cl_ref[...] = jnp.reshape(cls, (1, 1))
    ql_ref[...] = jnp.reshape(qua, (1, 1))


def _k3_call(Ub, Yb, S1, R1, W1, Bi, Ui, Yi):
    return pl.pallas_call(
        _k3_body,
        out_shape=[jax.ShapeDtypeStruct((1, 1), _F32)] * 4,
        scratch_shapes=[
            pltpu.VMEM((_BITS, _BATCH), _BF16),
        ],
        interpret=_INTERPRET,
    )(Ub, Yb, S1, R1, W1, Bi, Ui, Yi)


_NW = 32
_NCB = _BATCH // 16
_RPG = _BITS // (_NW // _NCB)


_RPGY = _C // (_NW // _NCB)


def _sc_gather(b_flat, u_flat, y_flat, indices):
    mesh = plsc.VectorSubcoreMesh(core_axis_name="c", subcore_axis_name="s")

    @functools.partial(
        pl.kernel,
        mesh=mesh,
        out_type=(
            jax.ShapeDtypeStruct((_BITS * _BATCH,), _F32),
            jax.ShapeDtypeStruct((_BITS * _BATCH,), _F32),
            jax.ShapeDtypeStruct((_C * _BATCH,), _F32),
        ),
        scratch_types=[
            pltpu.VMEM((16,), jnp.int32),
            pltpu.VMEM((_RPGY, 16), _F32),
            pltpu.SemaphoreType.DMA,
        ],
    )
    def k(b_hbm, u_hbm, y_hbm, idx_hbm, bi_hbm, ui_hbm, yi_hbm,
          idx_v, rbuf, sem):
        wid = lax.axis_index("s") * 2 + lax.axis_index("c")
        cb = wid % _NCB
        rg = wid // _NCB
        pltpu.sync_copy(idx_hbm.at[pl.ds(cb * 16, 16)], idx_v)
        ivec = idx_v[...]
        for src, dst, nrows in ((b_hbm, bi_hbm, _RPG), (u_hbm, ui_hbm, _RPG),
                                (y_hbm, yi_hbm, _RPGY)):
            r0 = rg * nrows
            for c0 in range(0, nrows, 13):
                rows = range(c0, min(c0 + 13, nrows))
                handles = []
                for r in rows:
                    flat = ivec + (r0 + r) * _N
                    handles.append(
                        pltpu.async_copy(src.at[flat], rbuf.at[r], sem))
                for h in handles:
                    h.wait()
            for r in range(nrows):
                pltpu.sync_copy(
                    rbuf.at[r],
                    dst.at[pl.ds((r0 + r) * _BATCH + cb * 16, 16)])

    return k(b_flat, u_flat, y_flat, indices)


def kernel(image_hash_features, onehot_labels, indices, B, U, Y):
    Bi, Ui, Yi = _sc_gather(B.reshape(-1), U.reshape(-1),
                            Y.reshape(-1), indices)
    S1, R1, W1 = _k12_call(B, Y, U)
    l, sl, cl, ql = _k3_call(image_hash_features, onehot_labels,
                             S1, R1, W1, Bi.reshape(_BITS, _BATCH),
                             Ui.reshape(_BITS, _BATCH),
                             Yi.reshape(_C, _BATCH))
    return (l[0, 0], sl[0, 0], cl[0, 0], ql[0, 0])
